# Initial kernel scaffold; baseline (speedup 1.0000x reference)
import jax
import jax.numpy as jnp
from jax.experimental import pallas as pl


def kernel(dense_input, sparse_input, emb1, emb2, fm_w, w1, w2, w3):
    raise NotImplementedError("write your pallas kernel here")


# speedup vs baseline: 23.7505x; 23.7505x over previous
import functools

import jax
import jax.numpy as jnp
import numpy as np
from jax import lax
from jax.experimental import pallas as pl
from jax.experimental.pallas import tpu as pltpu
from jax.experimental.pallas import tpu_sc as plsc

_B = 16384
_S = 26
_E = 16
_F = 4823
_FPAD = 4824
_NW = 32
_NPW = _B * _S // _NW
_CHUNK = 1024
_NCH = _NPW // _CHUNK
_RPW = _B // _NW
_TC_BB = 1024


def _sc_gather(idx_flat, emb1_pad, emb2):
    mesh = plsc.VectorSubcoreMesh(core_axis_name="c", subcore_axis_name="s")

    @functools.partial(
        pl.kernel,
        mesh=mesh,
        out_type=[
            jax.ShapeDtypeStruct((_B * _S, _E), jnp.float32),
            jax.ShapeDtypeStruct((_B * _S,), jnp.float32),
        ],
        scratch_types=[
            pltpu.VMEM((_NPW,), jnp.int32),
            pltpu.VMEM((_CHUNK, _E), jnp.float32),
            pltpu.VMEM((_CHUNK,), jnp.float32),
            pltpu.SemaphoreType.DMA,
            pltpu.SemaphoreType.DMA,
        ],
        compiler_params=pltpu.CompilerParams(use_tc_tiling_on_sc=False),
    )
    def k(idx_hbm, emb1_hbm, emb2_hbm, rows_out, vals1_out,
          idx_v, rows_v, vals1_v, sem, sem1):
        wid = lax.axis_index("s") * 2 + lax.axis_index("c")
        base = wid * _NPW
        pltpu.sync_copy(idx_hbm.at[pl.ds(base, _NPW)], idx_v)
        for c in range(_NCH):
            idx_c = idx_v.at[pl.ds(c * _CHUNK, _CHUNK)]
            cp2 = pltpu.async_copy(emb2_hbm.at[idx_c], rows_v, sem)
            cp1 = pltpu.async_copy(emb1_hbm.at[idx_c], vals1_v, sem1)
            cp2.wait()
            pltpu.sync_copy(rows_v,
                            rows_out.at[pl.ds(base + c * _CHUNK, _CHUNK)])
            cp1.wait()
            pltpu.sync_copy(vals1_v,
                            vals1_out.at[pl.ds(base + c * _CHUNK, _CHUNK)])

    return k(idx_flat, emb1_pad, emb2)


def _tc_body(e_ref, d_ref, e1_ref, fmw_ref, w1t_ref, w2t_ref, w3_ref,
             sel_ref, o_ref):
    e = e_ref[...]
    h1 = jnp.maximum(
        jnp.dot(e, w1t_ref[...], preferred_element_type=jnp.float32), 0.0)
    h2 = jnp.maximum(
        jnp.dot(h1, w2t_ref[...], preferred_element_type=jnp.float32), 0.0)
    y3 = jnp.sum(h2 * w3_ref[...], axis=1, keepdims=True)
    ssum = jnp.dot(e, sel_ref[...], preferred_element_type=jnp.float32)
    y2 = 0.5 * (jnp.sum(ssum * ssum, axis=1, keepdims=True)
                - jnp.sum(e * e, axis=1, keepdims=True))
    y1 = (jnp.sum(d_ref[...] * fmw_ref[...], axis=1, keepdims=True)
          + jnp.sum(e1_ref[...], axis=1, keepdims=True))
    z = y1 + y2 + y3
    o_ref[...] = 1.0 / (1.0 + jnp.exp(-z))


def _tc_mlp(e_flat, dense, e1, fm_w, w1t, w2t, w3, sel):
    return pl.pallas_call(
        _tc_body,
        grid=(_B // _TC_BB,),
        in_specs=[
            pl.BlockSpec((_TC_BB, _S * _E), lambda i: (i, 0)),
            pl.BlockSpec((_TC_BB, 13), lambda i: (i, 0)),
            pl.BlockSpec((_TC_BB, _S), lambda i: (i, 0)),
            pl.BlockSpec((1, 13), lambda i: (0, 0)),
            pl.BlockSpec((_S * _E, 256), lambda i: (0, 0)),
            pl.BlockSpec((256, 256), lambda i: (0, 0)),
            pl.BlockSpec((1, 256), lambda i: (0, 0)),
            pl.BlockSpec((_S * _E, _E), lambda i: (0, 0)),
        ],
        out_specs=pl.BlockSpec((_TC_BB, 1), lambda i: (i, 0)),
        out_shape=jax.ShapeDtypeStruct((_B, 1), jnp.float32),
    )(e_flat, dense, e1, fm_w, w1t, w2t, w3, sel)


_SEL = np.tile(np.eye(_E, dtype=np.float32), (_S, 1))


def kernel(dense_input, sparse_input, emb1, emb2, fm_w, w1, w2, w3):
    idx_flat = sparse_input.astype(jnp.int32).reshape(-1)
    rows, vals1 = _sc_gather(idx_flat, emb1.reshape(_F), emb2)
    e_flat = rows.reshape(_B, _S * _E)
    e1 = vals1.reshape(_B, _S)
    return _tc_mlp(e_flat, dense_input, e1, fm_w,
                   w1.T, w2.T, w3, jnp.asarray(_SEL))
